# baseline (speedup 1.0000x reference)
import functools

import jax
import jax.numpy as jnp
from jax import lax
from jax.experimental import pallas as pl
from jax.experimental.pallas import tpu as pltpu
from jax.experimental.pallas import tpu_sc as plsc

N = 10000
E = 320000
D = 128
H = 4
C = 16
HC = H * C
WROW = 80
TW = 80
T1W = 72
ZR = 48

NC = 2
NS = 16
NW = NC * NS
EPW = E // NW
K = 40
NCHUNK = EPW // K
ROWS_PT = 624
ROWS_REM = N - NS * ROWS_PT

BN = 1000
NB = N // BN

_f32 = jnp.float32


def _leaky(z):
    return jnp.where(z >= 0, z, z * jnp.float32(0.2))


def _proj_body(x_ref, wi_ref, wo_ref, asi_ref, adi_ref, aso_ref, ado_ref,
               t1i_ref, t1o_ref, as_ref, bmax_ref):
    x = x_ref[...]
    hi = jnp.dot(x, wi_ref[...], preferred_element_type=_f32)
    ho = jnp.dot(x, wo_ref[...], preferred_element_type=_f32)
    asi = jnp.dot(hi, asi_ref[...], preferred_element_type=_f32)
    adi = jnp.dot(hi, adi_ref[...], preferred_element_type=_f32)
    aso = jnp.dot(ho, aso_ref[...], preferred_element_type=_f32)
    ado = jnp.dot(ho, ado_ref[...], preferred_element_type=_f32)
    z4 = jnp.zeros_like(asi)
    t1i_ref[...] = jnp.concatenate([hi, asi, z4], axis=1)
    t1o_ref[...] = jnp.concatenate([ho, aso, z4], axis=1)
    as_ref[...] = jnp.concatenate([asi, adi, aso, ado], axis=1)
    bmax_ref[0] = jnp.concatenate(
        [jnp.max(asi, axis=0, keepdims=True), jnp.max(aso, axis=0, keepdims=True)],
        axis=1)


def _tables_body(t1i_ref, t1o_ref, as_ref, bmax_ref, r_ref,
                 ts_ref, td_ref, inii_ref, inio_ref):
    bm = bmax_ref[...]
    rmat = r_ref[...]
    gmax = jnp.max(bm, axis=0)
    asv = as_ref[...]
    asi, adi = asv[:, 0:4], asv[:, 4:8]
    aso, ado = asv[:, 8:12], asv[:, 12:16]
    ci = _leaky(adi + gmax[:, 0:4])
    co = _leaky(ado + gmax[:, 4:8])
    hi = t1i_ref[:, 0:HC]
    ho = t1o_ref[:, 0:HC]
    z4 = jnp.zeros_like(ci)
    ts_ref[...] = jnp.concatenate([hi, asi, z4, ado, co], axis=1)
    td_ref[...] = jnp.concatenate([ho, aso, z4, adi, ci], axis=1)

    def ini(h, ws, ini_ref):
        wide = jnp.dot(ws, rmat, preferred_element_type=_f32)
        ini_ref[...] = jnp.concatenate(
            [h * wide, ws, jnp.zeros((h.shape[0], WROW - HC - H), _f32)], axis=1)

    ini(hi, jnp.exp(_leaky(asi + adi) - ci), inii_ref)
    ini(ho, jnp.exp(_leaky(aso + ado) - co), inio_ref)


def _combine_body(pi_ref, po_ref, inii_ref, inio_ref, bi_ref, bo_ref, r_ref,
                  oi_ref, oo_ref):
    rmat = r_ref[...]

    def onedir(p, ini, b):
        s = p[0] + p[1] + ini
        num = s[:, 0:HC]
        den = s[:, HC:HC + H]
        recip = jnp.float32(1.0) / (den + jnp.float32(1e-16))
        return num * jnp.dot(recip, rmat, preferred_element_type=_f32) + b

    oi_ref[...] = onedir(pi_ref[...], inii_ref[...], bi_ref[...])
    oo_ref[...] = onedir(po_ref[...], inio_ref[...], bo_ref[...])


def _splat(v, i):
    idx = jnp.full((16,), i, dtype=jnp.int32)
    return lax.gather(
        v, idx[:, None],
        lax.GatherDimensionNumbers(offset_dims=(), collapsed_slice_dims=(0,),
                                   start_index_map=(0,)),
        (1,), mode=lax.GatherScatterMode.PROMISE_IN_BOUNDS)


_sc_mesh = plsc.VectorSubcoreMesh(core_axis_name="c", subcore_axis_name="s",
                                  num_cores=NC, num_subcores=NS)


@functools.partial(
    pl.kernel,
    out_type=jax.ShapeDtypeStruct((NC, 2, N, WROW), _f32),
    mesh=_sc_mesh,
    compiler_params=pltpu.CompilerParams(use_tc_tiling_on_sc=False,
                                         needs_layout_passes=False),
    scratch_types=[
        pltpu.VMEM((K,), jnp.int32), pltpu.VMEM((K,), jnp.int32),
        pltpu.VMEM((K, TW), _f32), pltpu.VMEM((K, TW), _f32),
        pltpu.VMEM((K, WROW), _f32), pltpu.VMEM((K, WROW), _f32),
        pltpu.SemaphoreType.DMA, pltpu.SemaphoreType.DMA,
        pltpu.VMEM((K,), jnp.int32), pltpu.VMEM((K,), jnp.int32),
        pltpu.VMEM((K, TW), _f32), pltpu.VMEM((K, TW), _f32),
        pltpu.VMEM((K, WROW), _f32), pltpu.VMEM((K, WROW), _f32),
        pltpu.SemaphoreType.DMA, pltpu.SemaphoreType.DMA,
        pltpu.VMEM((ZR, WROW), _f32),
        pltpu.VMEM_SHARED((N, WROW), _f32),
        pltpu.VMEM_SHARED((N, WROW), _f32),
    ],
)
def _sc_edges(ts, td, se, de, out,
              sidx_a, didx_a, sbuf_a, dbuf_a, ob0_a, ob1_a, isem_a, gsem_a,
              sidx_b, didx_b, sbuf_b, dbuf_b, ob0_b, ob1_b, isem_b, gsem_b,
              zbuf, acc0, acc1):
    buf_a = (sidx_a, didx_a, sbuf_a, dbuf_a, ob0_a, ob1_a, isem_a, gsem_a)
    buf_b = (sidx_b, didx_b, sbuf_b, dbuf_b, ob0_b, ob1_b, isem_b, gsem_b)
    cid = lax.axis_index("c")
    sid = lax.axis_index("s")
    wid = sid * NC + cid
    base = wid * EPW
    r0 = sid * ROWS_PT

    def zfill(rr, c2):
        for j in range(WROW // 16):
            zbuf[rr, pl.ds(j * 16, 16)] = jnp.zeros((16,), _f32)
        return c2

    lax.fori_loop(0, ZR, zfill, 0)

    def zcopy(t, c2):
        pltpu.sync_copy(zbuf, acc0.at[pl.ds(r0 + t * ZR, ZR)])
        pltpu.sync_copy(zbuf, acc1.at[pl.ds(r0 + t * ZR, ZR)])
        return c2

    lax.fori_loop(0, ROWS_PT // ZR, zcopy, 0)

    @pl.when(sid == NS - 1)
    def _():
        rr = NS * ROWS_PT
        pltpu.sync_copy(zbuf.at[pl.ds(0, ROWS_REM)], acc0.at[pl.ds(rr, ROWS_REM)])
        pltpu.sync_copy(zbuf.at[pl.ds(0, ROWS_REM)], acc1.at[pl.ds(rr, ROWS_REM)])

    plsc.subcore_barrier()

    lane = lax.iota(jnp.int32, 16)
    eoff = lane >> 2
    hsel = lane & 3

    hselA = hsel + HC
    hselB = hsel + HC + 8
    hselC = hsel + HC + 12
    hselw = hsel + HC

    def fire_idx(bufs, c):
        sidx, didx = bufs[0], bufs[1]
        isem = bufs[6]
        off = base + c * K
        pltpu.async_copy(se.at[pl.ds(off, K)], sidx, isem)
        pltpu.async_copy(de.at[pl.ds(off, K)], didx, isem)

    def wait_idx(bufs):
        sidx, didx = bufs[0], bufs[1]
        isem = bufs[6]
        pltpu.make_async_copy(se.at[pl.ds(0, K)], sidx, isem).wait()
        pltpu.make_async_copy(de.at[pl.ds(0, K)], didx, isem).wait()

    def fire_g(bufs):
        sidx, didx, sbuf, dbuf = bufs[0], bufs[1], bufs[2], bufs[3]
        gsem = bufs[7]
        pltpu.async_copy(ts.at[sidx], sbuf, gsem)
        pltpu.async_copy(td.at[didx], dbuf, gsem)

    def wait_g(bufs):
        sidx, didx, sbuf, dbuf = bufs[0], bufs[1], bufs[2], bufs[3]
        gsem = bufs[7]
        pltpu.make_async_copy(ts.at[sidx], sbuf, gsem).wait()
        pltpu.make_async_copy(td.at[didx], dbuf, gsem).wait()

    def compute_scatter(bufs):
        sidx, didx, sbuf, dbuf, ob0, ob1 = bufs[:6]

        @plsc.parallel_loop(0, K // 4, 1, unroll=2)
        def group_body(g):
            idx_e = g * 4 + eoff
            for srcb, dstb, obuf in ((sbuf, dbuf, ob0), (dbuf, sbuf, ob1)):
                v_as = plsc.load_gather(srcb, [idx_e, hselA])
                v_ad = plsc.load_gather(dstb, [idx_e, hselB])
                v_c = plsc.load_gather(dstb, [idx_e, hselC])
                w = jnp.exp(_leaky(v_as + v_ad) - v_c)
                plsc.store_scatter(obuf, [idx_e, hselw], w)
                for kk in range(4):
                    row = g * 4 + kk
                    for j in range(H):
                        sp = _splat(w, 4 * kk + j)
                        obuf[row, pl.ds(j * C, C)] = srcb[row, pl.ds(j * C, C)] * sp

        pltpu.sync_copy(ob0, acc0.at[didx], add=True)
        pltpu.sync_copy(ob1, acc1.at[sidx], add=True)

    def do_chunk(c, bufs, nxt, fire_next, idx2_pred):
        if fire_next:
            wait_idx(nxt)
            fire_g(nxt)
        wait_g(bufs)
        compute_scatter(bufs)
        if idx2_pred is True:
            fire_idx(bufs, c + 2)
        elif idx2_pred is not False:
            @pl.when(idx2_pred)
            def _():
                fire_idx(bufs, c + 2)

    fire_idx(buf_a, 0)
    wait_idx(buf_a)
    fire_g(buf_a)
    fire_idx(buf_b, 1)

    npairs = NCHUNK // 2 - 1

    def pair_body(j, carry):
        c = j * 2
        do_chunk(c, buf_a, buf_b, True, True)
        do_chunk(c + 1, buf_b, buf_a, True, True)
        return carry

    lax.fori_loop(0, npairs, pair_body, 0)
    do_chunk(NCHUNK - 2, buf_a, buf_b, True, False)
    do_chunk(NCHUNK - 1, buf_b, buf_a, False, False)

    plsc.subcore_barrier()
    pltpu.sync_copy(acc0.at[pl.ds(r0, ROWS_PT)], out.at[cid, 0, pl.ds(r0, ROWS_PT)])
    pltpu.sync_copy(acc1.at[pl.ds(r0, ROWS_PT)], out.at[cid, 1, pl.ds(r0, ROWS_PT)])

    @pl.when(sid == NS - 1)
    def _():
        rr = NS * ROWS_PT
        pltpu.sync_copy(acc0.at[pl.ds(rr, ROWS_REM)], out.at[cid, 0, pl.ds(rr, ROWS_REM)])
        pltpu.sync_copy(acc1.at[pl.ds(rr, ROWS_REM)], out.at[cid, 1, pl.ds(rr, ROWS_REM)])


def kernel(x, edge_index, W_in, att_src_in, att_dst_in, bias_in,
           W_out, att_src_out, att_dst_out, bias_out):
    mask = (jnp.arange(HC)[:, None] // C == jnp.arange(H)[None, :]).astype(_f32)
    a_si = att_src_in.reshape(HC, 1) * mask
    a_di = att_dst_in.reshape(HC, 1) * mask
    a_so = att_src_out.reshape(HC, 1) * mask
    a_do = att_dst_out.reshape(HC, 1) * mask
    rmat = mask.T

    full = lambda shp: pl.BlockSpec(shp, lambda i: tuple(0 for _ in shp))

    t1i, t1o, asv, bmax = pl.pallas_call(
        _proj_body,
        grid=(NB,),
        in_specs=[
            pl.BlockSpec((BN, D), lambda i: (i, 0)),
            full((D, HC)), full((D, HC)),
            full((HC, H)), full((HC, H)), full((HC, H)), full((HC, H)),
        ],
        out_specs=[
            pl.BlockSpec((BN, T1W), lambda i: (i, 0)),
            pl.BlockSpec((BN, T1W), lambda i: (i, 0)),
            pl.BlockSpec((BN, 16), lambda i: (i, 0)),
            pl.BlockSpec((1, 1, 8), lambda i: (i, 0, 0)),
        ],
        out_shape=[
            jax.ShapeDtypeStruct((N, T1W), _f32),
            jax.ShapeDtypeStruct((N, T1W), _f32),
            jax.ShapeDtypeStruct((N, 16), _f32),
            jax.ShapeDtypeStruct((NB, 1, 8), _f32),
        ],
    )(x, W_in, W_out, a_si, a_di, a_so, a_do)

    tsm, tdm, inii, inio = pl.pallas_call(
        _tables_body,
        grid=(NB,),
        in_specs=[
            pl.BlockSpec((BN, T1W), lambda i: (i, 0)),
            pl.BlockSpec((BN, T1W), lambda i: (i, 0)),
            pl.BlockSpec((BN, 16), lambda i: (i, 0)),
            full((NB, 1, 8)),
            full((H, HC)),
        ],
        out_specs=[
            pl.BlockSpec((BN, TW), lambda i: (i, 0)),
            pl.BlockSpec((BN, TW), lambda i: (i, 0)),
            pl.BlockSpec((BN, WROW), lambda i: (i, 0)),
            pl.BlockSpec((BN, WROW), lambda i: (i, 0)),
        ],
        out_shape=[
            jax.ShapeDtypeStruct((N, TW), _f32),
            jax.ShapeDtypeStruct((N, TW), _f32),
            jax.ShapeDtypeStruct((N, WROW), _f32),
            jax.ShapeDtypeStruct((N, WROW), _f32),
        ],
    )(t1i, t1o, asv, bmax, rmat)

    se = edge_index[0].astype(jnp.int32)
    de = edge_index[1].astype(jnp.int32)

    partial = _sc_edges(tsm, tdm, se, de)

    oi, oo = pl.pallas_call(
        _combine_body,
        grid=(NB,),
        in_specs=[
            pl.BlockSpec((NC, BN, WROW), lambda i: (0, i, 0)),
            pl.BlockSpec((NC, BN, WROW), lambda i: (0, i, 0)),
            pl.BlockSpec((BN, WROW), lambda i: (i, 0)),
            pl.BlockSpec((BN, WROW), lambda i: (i, 0)),
            full((1, HC)), full((1, HC)),
            full((H, HC)),
        ],
        out_specs=[
            pl.BlockSpec((BN, HC), lambda i: (i, 0)),
            pl.BlockSpec((BN, HC), lambda i: (i, 0)),
        ],
        out_shape=[
            jax.ShapeDtypeStruct((N, HC), _f32),
            jax.ShapeDtypeStruct((N, HC), _f32),
        ],
    )(partial[:, 0], partial[:, 1], inii, inio,
      bias_in.reshape(1, HC), bias_out.reshape(1, HC), rmat)

    return jnp.concatenate([oi, oo], axis=1)


# speedup vs baseline: 1.1109x; 1.1109x over previous
import functools

import jax
import jax.numpy as jnp
from jax import lax
from jax.experimental import pallas as pl
from jax.experimental.pallas import tpu as pltpu
from jax.experimental.pallas import tpu_sc as plsc

N = 10000
E = 320000
D = 128
H = 4
C = 16
HC = H * C
WROW = 72
TW = 80
T1W = 72

NC = 2
NS = 16
NW = NC * NS
EPW = E // NW
K = 80
NCHUNK = EPW // K
ROWS_PT = 624
ROWS_REM = N - NS * ROWS_PT

BN = 1000
NB = N // BN

_f32 = jnp.float32


def _leaky(z):
    return jnp.where(z >= 0, z, z * jnp.float32(0.2))


def _proj_body(x_ref, wi_ref, wo_ref, asi_ref, adi_ref, aso_ref, ado_ref,
               t1i_ref, t1o_ref, as_ref, bmax_ref):
    x = x_ref[...]
    hi = jnp.dot(x, wi_ref[...], preferred_element_type=_f32)
    ho = jnp.dot(x, wo_ref[...], preferred_element_type=_f32)
    asi = jnp.dot(hi, asi_ref[...], preferred_element_type=_f32)
    adi = jnp.dot(hi, adi_ref[...], preferred_element_type=_f32)
    aso = jnp.dot(ho, aso_ref[...], preferred_element_type=_f32)
    ado = jnp.dot(ho, ado_ref[...], preferred_element_type=_f32)
    z4 = jnp.zeros_like(asi)
    t1i_ref[...] = jnp.concatenate([hi, asi, z4], axis=1)
    t1o_ref[...] = jnp.concatenate([ho, aso, z4], axis=1)
    as_ref[...] = jnp.concatenate([asi, adi, aso, ado], axis=1)
    bmax_ref[0] = jnp.concatenate(
        [jnp.max(asi, axis=0, keepdims=True), jnp.max(aso, axis=0, keepdims=True)],
        axis=1)


def _tables_body(t1i_ref, t1o_ref, as_ref, bmax_ref, r_ref,
                 ts_ref, td_ref, inii_ref, inio_ref):
    bm = bmax_ref[...]
    rmat = r_ref[...]
    gmax = jnp.max(bm, axis=0)
    asv = as_ref[...]
    asi, adi = asv[:, 0:4], asv[:, 4:8]
    aso, ado = asv[:, 8:12], asv[:, 12:16]
    ci = _leaky(adi + gmax[:, 0:4])
    co = _leaky(ado + gmax[:, 4:8])
    hi = t1i_ref[:, 0:HC]
    ho = t1o_ref[:, 0:HC]
    z4 = jnp.zeros_like(ci)
    ts_ref[...] = jnp.concatenate([hi, asi, z4, ado, co], axis=1)
    td_ref[...] = jnp.concatenate([ho, aso, z4, adi, ci], axis=1)

    def ini(h, ws, ini_ref):
        wide = jnp.dot(ws, rmat, preferred_element_type=_f32)
        ini_ref[...] = jnp.concatenate(
            [h * wide, ws, jnp.zeros((h.shape[0], WROW - HC - H), _f32)], axis=1)

    ini(hi, jnp.exp(_leaky(asi + adi) - ci), inii_ref)
    ini(ho, jnp.exp(_leaky(aso + ado) - co), inio_ref)


def _combine_body(pi_ref, po_ref, inii_ref, inio_ref, bi_ref, bo_ref, r_ref,
                  oi_ref, oo_ref):
    rmat = r_ref[...]

    def onedir(p, ini, b):
        s = p[0] + p[1] + ini
        num = s[:, 0:HC]
        den = s[:, HC:HC + H]
        recip = jnp.float32(1.0) / (den + jnp.float32(1e-16))
        return num * jnp.dot(recip, rmat, preferred_element_type=_f32) + b

    oi_ref[...] = onedir(pi_ref[...], inii_ref[...], bi_ref[...])
    oo_ref[...] = onedir(po_ref[...], inio_ref[...], bo_ref[...])


def _splat(v, i):
    idx = jnp.full((16,), i, dtype=jnp.int32)
    return lax.gather(
        v, idx[:, None],
        lax.GatherDimensionNumbers(offset_dims=(), collapsed_slice_dims=(0,),
                                   start_index_map=(0,)),
        (1,), mode=lax.GatherScatterMode.PROMISE_IN_BOUNDS)


_sc_mesh = plsc.VectorSubcoreMesh(core_axis_name="c", subcore_axis_name="s",
                                  num_cores=NC, num_subcores=NS)


@functools.partial(
    pl.kernel,
    out_type=jax.ShapeDtypeStruct((NC, 2, N, WROW), _f32),
    mesh=_sc_mesh,
    compiler_params=pltpu.CompilerParams(use_tc_tiling_on_sc=False,
                                         needs_layout_passes=False),
    scratch_types=[
        pltpu.VMEM((K,), jnp.int32), pltpu.VMEM((K,), jnp.int32),
        pltpu.VMEM((K, TW), _f32), pltpu.VMEM((K, TW), _f32),
        pltpu.VMEM((K, WROW), _f32),
        pltpu.SemaphoreType.DMA, pltpu.SemaphoreType.DMA,
        pltpu.VMEM((K,), jnp.int32), pltpu.VMEM((K,), jnp.int32),
        pltpu.VMEM((K, TW), _f32), pltpu.VMEM((K, TW), _f32),
        pltpu.VMEM((K, WROW), _f32),
        pltpu.SemaphoreType.DMA, pltpu.SemaphoreType.DMA,
        pltpu.VMEM_SHARED((N, WROW), _f32),
        pltpu.VMEM_SHARED((N, WROW), _f32),
    ],
)
def _sc_edges(ts, td, se, de, zrow, out,
              sidx_a, didx_a, sbuf_a, dbuf_a, ob_a, isem_a, gsem_a,
              sidx_b, didx_b, sbuf_b, dbuf_b, ob_b, isem_b, gsem_b,
              acc0, acc1):
    buf_a = (sidx_a, didx_a, sbuf_a, dbuf_a, ob_a, isem_a, gsem_a)
    buf_b = (sidx_b, didx_b, sbuf_b, dbuf_b, ob_b, isem_b, gsem_b)
    cid = lax.axis_index("c")
    sid = lax.axis_index("s")
    wid = sid * NC + cid
    base = wid * EPW
    r0 = sid * ROWS_PT

    pltpu.sync_copy(zrow.at[pl.ds(r0, ROWS_PT)], acc0.at[pl.ds(r0, ROWS_PT)])
    pltpu.sync_copy(zrow.at[pl.ds(r0, ROWS_PT)], acc1.at[pl.ds(r0, ROWS_PT)])

    @pl.when(sid == NS - 1)
    def _():
        rr = NS * ROWS_PT
        pltpu.sync_copy(zrow.at[pl.ds(rr, ROWS_REM)], acc0.at[pl.ds(rr, ROWS_REM)])
        pltpu.sync_copy(zrow.at[pl.ds(rr, ROWS_REM)], acc1.at[pl.ds(rr, ROWS_REM)])

    plsc.subcore_barrier()

    lane = lax.iota(jnp.int32, 16)
    eoff = lane >> 2
    hsel = lane & 3

    hselA = hsel + HC
    hselB = hsel + HC + 8
    hselC = hsel + HC + 12
    hselw = hsel + HC

    def fire_idx(bufs, c):
        sidx, didx = bufs[0], bufs[1]
        isem = bufs[5]
        off = base + c * K
        pltpu.async_copy(se.at[pl.ds(off, K)], sidx, isem)
        pltpu.async_copy(de.at[pl.ds(off, K)], didx, isem)

    def wait_idx(bufs):
        sidx, didx = bufs[0], bufs[1]
        isem = bufs[5]
        pltpu.make_async_copy(se.at[pl.ds(0, K)], sidx, isem).wait()
        pltpu.make_async_copy(de.at[pl.ds(0, K)], didx, isem).wait()

    def fire_g(bufs):
        sidx, didx, sbuf, dbuf = bufs[0], bufs[1], bufs[2], bufs[3]
        gsem = bufs[6]
        pltpu.async_copy(ts.at[sidx], sbuf, gsem)
        pltpu.async_copy(td.at[didx], dbuf, gsem)

    def wait_g(bufs):
        sidx, didx, sbuf, dbuf = bufs[0], bufs[1], bufs[2], bufs[3]
        gsem = bufs[6]
        pltpu.make_async_copy(ts.at[sidx], sbuf, gsem).wait()
        pltpu.make_async_copy(td.at[didx], dbuf, gsem).wait()

    def compute_scatter(bufs):
        sidx, didx, sbuf, dbuf, ob = bufs[:5]

        def onedir(srcb, dstb):
            @plsc.parallel_loop(0, K // 4, 1, unroll=4)
            def group_body(g):
                idx_e = g * 4 + eoff
                v_as = plsc.load_gather(srcb, [idx_e, hselA])
                v_ad = plsc.load_gather(dstb, [idx_e, hselB])
                v_c = plsc.load_gather(dstb, [idx_e, hselC])
                w = jnp.exp(_leaky(v_as + v_ad) - v_c)
                plsc.store_scatter(ob, [idx_e, hselw], w)
                for kk in range(4):
                    row = g * 4 + kk
                    for j in range(H):
                        sp = _splat(w, 4 * kk + j)
                        ob[row, pl.ds(j * C, C)] = srcb[row, pl.ds(j * C, C)] * sp

        onedir(sbuf, dbuf)
        pltpu.sync_copy(ob, acc0.at[didx], add=True)
        onedir(dbuf, sbuf)
        pltpu.sync_copy(ob, acc1.at[sidx], add=True)

    def do_chunk(c, bufs, nxt, fire_next, idx2_pred):
        if fire_next:
            wait_idx(nxt)
            fire_g(nxt)
        wait_g(bufs)
        compute_scatter(bufs)
        if idx2_pred is True:
            fire_idx(bufs, c + 2)
        elif idx2_pred is not False:
            @pl.when(idx2_pred)
            def _():
                fire_idx(bufs, c + 2)

    fire_idx(buf_a, 0)
    wait_idx(buf_a)
    fire_g(buf_a)
    fire_idx(buf_b, 1)

    npairs = (NCHUNK - 1) // 2

    def pair_body(j, carry):
        c = j * 2
        do_chunk(c, buf_a, buf_b, True, True)
        do_chunk(c + 1, buf_b, buf_a, True, j < npairs - 1)
        return carry

    lax.fori_loop(0, npairs, pair_body, 0)
    do_chunk(NCHUNK - 1, buf_a, buf_b, False, False)

    plsc.subcore_barrier()
    pltpu.sync_copy(acc0.at[pl.ds(r0, ROWS_PT)], out.at[cid, 0, pl.ds(r0, ROWS_PT)])
    pltpu.sync_copy(acc1.at[pl.ds(r0, ROWS_PT)], out.at[cid, 1, pl.ds(r0, ROWS_PT)])

    @pl.when(sid == NS - 1)
    def _():
        rr = NS * ROWS_PT
        pltpu.sync_copy(acc0.at[pl.ds(rr, ROWS_REM)], out.at[cid, 0, pl.ds(rr, ROWS_REM)])
        pltpu.sync_copy(acc1.at[pl.ds(rr, ROWS_REM)], out.at[cid, 1, pl.ds(rr, ROWS_REM)])


def kernel(x, edge_index, W_in, att_src_in, att_dst_in, bias_in,
           W_out, att_src_out, att_dst_out, bias_out):
    mask = (jnp.arange(HC)[:, None] // C == jnp.arange(H)[None, :]).astype(_f32)
    a_si = att_src_in.reshape(HC, 1) * mask
    a_di = att_dst_in.reshape(HC, 1) * mask
    a_so = att_src_out.reshape(HC, 1) * mask
    a_do = att_dst_out.reshape(HC, 1) * mask
    rmat = mask.T

    full = lambda shp: pl.BlockSpec(shp, lambda i: tuple(0 for _ in shp))

    t1i, t1o, asv, bmax = pl.pallas_call(
        _proj_body,
        grid=(NB,),
        in_specs=[
            pl.BlockSpec((BN, D), lambda i: (i, 0)),
            full((D, HC)), full((D, HC)),
            full((HC, H)), full((HC, H)), full((HC, H)), full((HC, H)),
        ],
        out_specs=[
            pl.BlockSpec((BN, T1W), lambda i: (i, 0)),
            pl.BlockSpec((BN, T1W), lambda i: (i, 0)),
            pl.BlockSpec((BN, 16), lambda i: (i, 0)),
            pl.BlockSpec((1, 1, 8), lambda i: (i, 0, 0)),
        ],
        out_shape=[
            jax.ShapeDtypeStruct((N, T1W), _f32),
            jax.ShapeDtypeStruct((N, T1W), _f32),
            jax.ShapeDtypeStruct((N, 16), _f32),
            jax.ShapeDtypeStruct((NB, 1, 8), _f32),
        ],
    )(x, W_in, W_out, a_si, a_di, a_so, a_do)

    tsm, tdm, inii, inio = pl.pallas_call(
        _tables_body,
        grid=(NB,),
        in_specs=[
            pl.BlockSpec((BN, T1W), lambda i: (i, 0)),
            pl.BlockSpec((BN, T1W), lambda i: (i, 0)),
            pl.BlockSpec((BN, 16), lambda i: (i, 0)),
            full((NB, 1, 8)),
            full((H, HC)),
        ],
        out_specs=[
            pl.BlockSpec((BN, TW), lambda i: (i, 0)),
            pl.BlockSpec((BN, TW), lambda i: (i, 0)),
            pl.BlockSpec((BN, WROW), lambda i: (i, 0)),
            pl.BlockSpec((BN, WROW), lambda i: (i, 0)),
        ],
        out_shape=[
            jax.ShapeDtypeStruct((N, TW), _f32),
            jax.ShapeDtypeStruct((N, TW), _f32),
            jax.ShapeDtypeStruct((N, WROW), _f32),
            jax.ShapeDtypeStruct((N, WROW), _f32),
        ],
    )(t1i, t1o, asv, bmax, rmat)

    se = edge_index[0].astype(jnp.int32)
    de = edge_index[1].astype(jnp.int32)
    zrow = jnp.zeros((N, WROW), _f32)

    partial = _sc_edges(tsm, tdm, se, de, zrow)

    oi, oo = pl.pallas_call(
        _combine_body,
        grid=(NB,),
        in_specs=[
            pl.BlockSpec((NC, BN, WROW), lambda i: (0, i, 0)),
            pl.BlockSpec((NC, BN, WROW), lambda i: (0, i, 0)),
            pl.BlockSpec((BN, WROW), lambda i: (i, 0)),
            pl.BlockSpec((BN, WROW), lambda i: (i, 0)),
            full((1, HC)), full((1, HC)),
            full((H, HC)),
        ],
        out_specs=[
            pl.BlockSpec((BN, HC), lambda i: (i, 0)),
            pl.BlockSpec((BN, HC), lambda i: (i, 0)),
        ],
        out_shape=[
            jax.ShapeDtypeStruct((N, HC), _f32),
            jax.ShapeDtypeStruct((N, HC), _f32),
        ],
    )(partial[:, 0], partial[:, 1], inii, inio,
      bias_in.reshape(1, HC), bias_out.reshape(1, HC), rmat)

    return jnp.concatenate([oi, oo], axis=1)
